# Initial kernel scaffold; baseline (speedup 1.0000x reference)
import functools

import jax
import jax.numpy as jnp
from jax import lax
from jax.experimental import pallas as pl
from jax.experimental.pallas import tpu as pltpu
from jax.experimental.pallas import tpu_sc as plsc

NN = 10000
NP = 10240
DH = 32
CHUNK = 128
NCH = 79
NSUB = 16
NCORE = 2
EPC = NSUB * NCH * CHUNK
E2 = NCORE * EPC
RPT = NP // NSUB

_mesh = plsc.VectorSubcoreMesh(core_axis_name="c", subcore_axis_name="s")


@functools.partial(
    pl.kernel,
    mesh=_mesh,
    out_type=jax.ShapeDtypeStruct((NCORE, NP), jnp.float32),
    scratch_types=[
        pltpu.VMEM((NCH, CHUNK), jnp.int32),
        pltpu.VMEM((CHUNK,), jnp.float32),
        pltpu.VMEM_SHARED((NP,), jnp.float32),
    ],
)
def _sc_deg(dst_hbm, zeros_hbm, out_hbm, didx, ones, acc):
    c = lax.axis_index("c")
    s = lax.axis_index("s")
    r0 = s * RPT
    pltpu.sync_copy(zeros_hbm.at[pl.ds(r0, RPT)], acc.at[pl.ds(r0, RPT)])
    pltpu.sync_copy(dst_hbm.at[c, s], didx)
    for k in range(CHUNK // 16):
        ones[pl.ds(k * 16, 16)] = jnp.ones((16,), jnp.float32)
    plsc.subcore_barrier()

    def body(j, carry):
        pltpu.sync_copy(ones, acc.at[didx.at[j]], add=True)
        return carry

    lax.fori_loop(0, NCH, body, 0)
    plsc.subcore_barrier()
    pltpu.sync_copy(acc.at[pl.ds(r0, RPT)], out_hbm.at[c, pl.ds(r0, RPT)])


@functools.partial(
    pl.kernel,
    mesh=_mesh,
    out_type=jax.ShapeDtypeStruct((NCORE, NP, DH), jnp.float32),
    scratch_types=[
        pltpu.VMEM((NCH, CHUNK), jnp.int32),
        pltpu.VMEM((NCH, CHUNK), jnp.int32),
        pltpu.VMEM((CHUNK, DH), jnp.float32),
        pltpu.VMEM_SHARED((NP, DH), jnp.float32),
        pltpu.SemaphoreType.DMA,
    ],
)
def _sc_agg(src_hbm, dst_hbm, table_hbm, zeros_hbm, out_hbm,
            sidx, didx, rows, acc, sem):
    c = lax.axis_index("c")
    s = lax.axis_index("s")
    r0 = s * RPT
    pltpu.sync_copy(zeros_hbm.at[pl.ds(r0, RPT)], acc.at[pl.ds(r0, RPT)])
    pltpu.sync_copy(src_hbm.at[c, s], sidx)
    pltpu.sync_copy(dst_hbm.at[c, s], didx)
    plsc.subcore_barrier()

    def body(j, carry):
        pltpu.async_copy(table_hbm.at[sidx.at[j]], rows, sem).wait()
        pltpu.sync_copy(rows, acc.at[didx.at[j]], add=True)
        return carry

    lax.fori_loop(0, NCH, body, 0)
    plsc.subcore_barrier()
    pltpu.sync_copy(acc.at[pl.ds(r0, RPT)], out_hbm.at[c, pl.ds(r0, RPT)])


def _tc_h1_body(x_ref, w_ref, h_ref):
    h = jnp.dot(x_ref[...], w_ref[...], preferred_element_type=jnp.float32)
    h_ref[pl.ds(0, NN), :] = h
    h_ref[pl.ds(NN, NP - NN), :] = jnp.zeros((NP - NN, DH), jnp.float32)


_tc_h1 = pl.pallas_call(
    _tc_h1_body,
    out_shape=jax.ShapeDtypeStruct((NP, DH), jnp.float32),
)


def _tc_scale_body(deg0_ref, deg1_ref, h_ref, dis_ref, hs_ref):
    deg = deg0_ref[...] + deg1_ref[...] + 1.0
    dis = lax.rsqrt(deg)
    dis_ref[...] = dis
    hs_ref[...] = h_ref[...] * dis


_tc_scale = pl.pallas_call(
    _tc_scale_body,
    out_shape=(
        jax.ShapeDtypeStruct((NP, 1), jnp.float32),
        jax.ShapeDtypeStruct((NP, DH), jnp.float32),
    ),
)


def _tc_mid_body(ga_ref, gb_ref, hs_ref, dis_ref, b_ref, w_ref, out_ref):
    dis = dis_ref[...]
    u = jnp.maximum(dis * (ga_ref[...] + gb_ref[...] + hs_ref[...]) + b_ref[...], 0.0)
    hs2 = jnp.dot(u, w_ref[...], preferred_element_type=jnp.float32) * dis
    out_ref[...] = hs2
    out_ref[pl.ds(NN, NP - NN), :] = jnp.zeros((NP - NN, DH), jnp.float32)


_tc_mid = pl.pallas_call(
    _tc_mid_body,
    out_shape=jax.ShapeDtypeStruct((NP, DH), jnp.float32),
)


def _tc_out_body(ga_ref, gb_ref, hs_ref, dis_ref, b_ref, wl_ref, bl_ref, out_ref):
    dis = dis_ref[...]
    v = jnp.maximum(dis * (ga_ref[...] + gb_ref[...] + hs_ref[...]) + b_ref[...], 0.0)
    out = jnp.dot(v, wl_ref[...], preferred_element_type=jnp.float32) + bl_ref[...]
    out_ref[...] = lax.slice(out, (0, 0), (NN, 1))


_tc_out = pl.pallas_call(
    _tc_out_body,
    out_shape=jax.ShapeDtypeStruct((NN, 1), jnp.float32),
)


def kernel(x, edge_index, W1, b1, W2, b2, Wl, bl):
    src = edge_index[0]
    dst = edge_index[1]
    e = src.shape[0]
    pad = jnp.full((E2 - e,), NN, dtype=jnp.int32)
    srcp = jnp.concatenate([src, pad]).reshape(NCORE, NSUB, NCH, CHUNK)
    dstp = jnp.concatenate([dst, pad]).reshape(NCORE, NSUB, NCH, CHUNK)
    zeros1 = jnp.zeros((NP,), jnp.float32)
    zeros2 = jnp.zeros((NP, DH), jnp.float32)

    degp = _sc_deg(dstp, zeros1)
    h1 = _tc_h1(x, W1)
    dis, hs1 = _tc_scale(degp[0].reshape(NP, 1), degp[1].reshape(NP, 1), h1)
    g1 = _sc_agg(srcp, dstp, hs1, zeros2)
    hs2 = _tc_mid(g1[0], g1[1], hs1, dis, b1.reshape(1, DH), W2)
    g2 = _sc_agg(srcp, dstp, hs2, zeros2)
    out = _tc_out(g2[0], g2[1], hs2, dis, b2.reshape(1, DH), Wl, bl.reshape(1, 1))
    return out


# speedup vs baseline: 26.5551x; 26.5551x over previous
import functools

import jax
import jax.numpy as jnp
from jax import lax
from jax.experimental import pallas as pl
from jax.experimental.pallas import tpu as pltpu
from jax.experimental.pallas import tpu_sc as plsc

NN = 10000
NP = 10240
DH = 32
CHUNK = 128
NCH = 79
NSUB = 16
NCORE = 2
EPC = NSUB * NCH * CHUNK
E2 = NCORE * EPC
RPT = NP // NSUB

@functools.cache
def _sc_kernels():
    mesh = plsc.VectorSubcoreMesh(core_axis_name="c", subcore_axis_name="s")

    @functools.partial(
        pl.kernel,
        mesh=mesh,
        out_type=jax.ShapeDtypeStruct((NCORE, NP), jnp.float32),
        scratch_types=[
            pltpu.VMEM((NCH, CHUNK), jnp.int32),
            pltpu.VMEM((CHUNK,), jnp.float32),
            pltpu.VMEM_SHARED((NP,), jnp.float32),
        ],
    )
    def _sc_deg(dst_hbm, zeros_hbm, out_hbm, didx, ones, acc):
        c = lax.axis_index("c")
        s = lax.axis_index("s")
        r0 = s * RPT
        pltpu.sync_copy(zeros_hbm.at[pl.ds(r0, RPT)], acc.at[pl.ds(r0, RPT)])
        pltpu.sync_copy(dst_hbm.at[c, s], didx)
        for k in range(CHUNK // 16):
            ones[pl.ds(k * 16, 16)] = jnp.ones((16,), jnp.float32)
        plsc.subcore_barrier()

        def body(j, carry):
            pltpu.sync_copy(ones, acc.at[didx.at[j]], add=True)
            return carry

        lax.fori_loop(0, NCH, body, 0)
        plsc.subcore_barrier()
        pltpu.sync_copy(acc.at[pl.ds(r0, RPT)], out_hbm.at[c, pl.ds(r0, RPT)])

    @functools.partial(
        pl.kernel,
        mesh=mesh,
        out_type=jax.ShapeDtypeStruct((NCORE, NP, DH), jnp.float32),
        compiler_params=pltpu.CompilerParams(use_tc_tiling_on_sc=False),
        scratch_types=[
            pltpu.VMEM((NCH, CHUNK), jnp.int32),
            pltpu.VMEM((NCH, CHUNK), jnp.int32),
            pltpu.VMEM((CHUNK, DH), jnp.float32),
            pltpu.VMEM_SHARED((NP, DH), jnp.float32),
            pltpu.SemaphoreType.DMA,
        ],
    )
    def _sc_agg(src_hbm, dst_hbm, table_hbm, zeros_hbm, out_hbm,
                sidx, didx, rows, acc, sem):
        c = lax.axis_index("c")
        s = lax.axis_index("s")
        r0 = s * RPT
        pltpu.sync_copy(zeros_hbm.at[pl.ds(r0, RPT)], acc.at[pl.ds(r0, RPT)])
        pltpu.sync_copy(src_hbm.at[c, s], sidx)
        pltpu.sync_copy(dst_hbm.at[c, s], didx)
        plsc.subcore_barrier()

        def body(j, carry):
            pltpu.async_copy(table_hbm.at[sidx.at[j]], rows, sem).wait()
            pltpu.sync_copy(rows, acc.at[didx.at[j]], add=True)
            return carry

        lax.fori_loop(0, NCH, body, 0)
        plsc.subcore_barrier()
        pltpu.sync_copy(acc.at[pl.ds(r0, RPT)], out_hbm.at[c, pl.ds(r0, RPT)])

    return _sc_deg, _sc_agg


def _tc_h1_body(x_ref, w_ref, h_ref):
    h = jnp.dot(x_ref[...], w_ref[...], preferred_element_type=jnp.float32)
    h_ref[pl.ds(0, NN), :] = h
    h_ref[pl.ds(NN, NP - NN), :] = jnp.zeros((NP - NN, DH), jnp.float32)


_tc_h1 = pl.pallas_call(
    _tc_h1_body,
    out_shape=jax.ShapeDtypeStruct((NP, DH), jnp.float32),
)


def _tc_scale_body(deg0_ref, deg1_ref, h_ref, dis_ref, hs_ref):
    deg = deg0_ref[...] + deg1_ref[...] + 1.0
    dis = lax.rsqrt(deg)
    dis_ref[...] = dis
    hs_ref[...] = h_ref[...] * dis


_tc_scale = pl.pallas_call(
    _tc_scale_body,
    out_shape=(
        jax.ShapeDtypeStruct((NP, 1), jnp.float32),
        jax.ShapeDtypeStruct((NP, DH), jnp.float32),
    ),
)


def _tc_mid_body(ga_ref, gb_ref, hs_ref, dis_ref, b_ref, w_ref, out_ref):
    dis = dis_ref[...]
    u = jnp.maximum(dis * (ga_ref[...] + gb_ref[...] + hs_ref[...]) + b_ref[...], 0.0)
    hs2 = jnp.dot(u, w_ref[...], preferred_element_type=jnp.float32) * dis
    out_ref[...] = hs2
    out_ref[pl.ds(NN, NP - NN), :] = jnp.zeros((NP - NN, DH), jnp.float32)


_tc_mid = pl.pallas_call(
    _tc_mid_body,
    out_shape=jax.ShapeDtypeStruct((NP, DH), jnp.float32),
)


def _tc_out_body(ga_ref, gb_ref, hs_ref, dis_ref, b_ref, wl_ref, bl_ref, out_ref):
    dis = dis_ref[...]
    v = jnp.maximum(dis * (ga_ref[...] + gb_ref[...] + hs_ref[...]) + b_ref[...], 0.0)
    out = jnp.dot(v, wl_ref[...], preferred_element_type=jnp.float32) + bl_ref[...]
    out_ref[...] = lax.slice(out, (0, 0), (NN, 1))


_tc_out = pl.pallas_call(
    _tc_out_body,
    out_shape=jax.ShapeDtypeStruct((NN, 1), jnp.float32),
)


def kernel(x, edge_index, W1, b1, W2, b2, Wl, bl):
    src = edge_index[0]
    dst = edge_index[1]
    e = src.shape[0]
    pad = jnp.full((E2 - e,), NN, dtype=jnp.int32)
    srcp = jnp.concatenate([src, pad]).reshape(NCORE, NSUB, NCH, CHUNK)
    dstp = jnp.concatenate([dst, pad]).reshape(NCORE, NSUB, NCH, CHUNK)
    zeros1 = jnp.zeros((NP,), jnp.float32)
    zeros2 = jnp.zeros((NP, DH), jnp.float32)

    _sc_deg, _sc_agg = _sc_kernels()
    degp = _sc_deg(dstp, zeros1)
    h1 = _tc_h1(x, W1)
    dis, hs1 = _tc_scale(degp[0].reshape(NP, 1), degp[1].reshape(NP, 1), h1)
    g1 = _sc_agg(srcp, dstp, hs1, zeros2)
    hs2 = _tc_mid(g1[0], g1[1], hs1, dis, b1.reshape(1, DH), W2)
    g2 = _sc_agg(srcp, dstp, hs2, zeros2)
    out = _tc_out(g2[0], g2[1], hs2, dis, b2.reshape(1, DH), Wl, bl.reshape(1, 1))
    return out


# speedup vs baseline: 27.0701x; 1.0194x over previous
import functools

import jax
import jax.numpy as jnp
from jax import lax
from jax.experimental import pallas as pl
from jax.experimental.pallas import tpu as pltpu
from jax.experimental.pallas import tpu_sc as plsc

NN = 10000
NP = 10240
DH = 32
CHUNK = 128
NCH = 80
NBUF = 8
NSUB = 16
NCORE = 2
EPC = NSUB * NCH * CHUNK
E2 = NCORE * EPC
RPT = NP // NSUB

@functools.cache
def _sc_kernels():
    mesh = plsc.VectorSubcoreMesh(core_axis_name="c", subcore_axis_name="s")

    @functools.partial(
        pl.kernel,
        mesh=mesh,
        out_type=jax.ShapeDtypeStruct((NCORE, NP), jnp.float32),
        scratch_types=[
            pltpu.VMEM((NCH, CHUNK), jnp.int32),
            pltpu.VMEM((CHUNK,), jnp.float32),
            pltpu.VMEM_SHARED((NP,), jnp.float32),
            pltpu.SemaphoreType.DMA,
        ],
    )
    def _sc_deg(dst_hbm, zeros_hbm, out_hbm, didx, ones, acc, ssem):
        c = lax.axis_index("c")
        s = lax.axis_index("s")
        r0 = s * RPT
        pltpu.sync_copy(zeros_hbm.at[pl.ds(r0, RPT)], acc.at[pl.ds(r0, RPT)])
        pltpu.sync_copy(dst_hbm.at[c, s], didx)
        for k in range(CHUNK // 16):
            ones[pl.ds(k * 16, 16)] = jnp.ones((16,), jnp.float32)
        plsc.subcore_barrier()

        def body(j, carry):
            pltpu.async_copy(ones, acc.at[didx.at[j]], ssem, add=True)
            return carry

        lax.fori_loop(0, NCH, body, 0)

        def drain(j, carry):
            pltpu.make_async_copy(ones, acc.at[didx.at[j]], ssem).wait()
            return carry

        lax.fori_loop(0, NCH, drain, 0)
        plsc.subcore_barrier()
        pltpu.sync_copy(acc.at[pl.ds(r0, RPT)], out_hbm.at[c, pl.ds(r0, RPT)])

    @functools.partial(
        pl.kernel,
        mesh=mesh,
        out_type=jax.ShapeDtypeStruct((NCORE, NP, DH), jnp.float32),
        compiler_params=pltpu.CompilerParams(use_tc_tiling_on_sc=False),
        scratch_types=[
            pltpu.VMEM((NCH, CHUNK), jnp.int32),
            pltpu.VMEM((NCH, CHUNK), jnp.int32),
            pltpu.VMEM((NBUF, CHUNK, DH), jnp.float32),
            pltpu.VMEM_SHARED((NP, DH), jnp.float32),
            pltpu.SemaphoreType.DMA((NBUF,)),
            pltpu.SemaphoreType.DMA((NBUF,)),
        ],
    )
    def _sc_agg(src_hbm, dst_hbm, table_hbm, zeros_hbm, out_hbm,
                sidx, didx, rows, acc, gsem, ssem):
        c = lax.axis_index("c")
        s = lax.axis_index("s")
        r0 = s * RPT
        pltpu.sync_copy(zeros_hbm.at[pl.ds(r0, RPT)], acc.at[pl.ds(r0, RPT)])
        pltpu.sync_copy(src_hbm.at[c, s], sidx)
        pltpu.sync_copy(dst_hbm.at[c, s], didx)
        plsc.subcore_barrier()

        def gather_start(j, b):
            pltpu.async_copy(table_hbm.at[sidx.at[j]], rows.at[b], gsem.at[b])

        def gather_wait(j, b):
            pltpu.make_async_copy(table_hbm.at[sidx.at[j]], rows.at[b],
                                  gsem.at[b]).wait()

        def scatter_start(j, b):
            pltpu.async_copy(rows.at[b], acc.at[didx.at[j]], ssem.at[b],
                             add=True)

        def scatter_wait(j, b):
            pltpu.make_async_copy(rows.at[b], acc.at[didx.at[j]],
                                  ssem.at[b]).wait()

        for b in range(NBUF):
            gather_start(b, b)

        ngrp = NCH // NBUF

        def body(i, carry):
            base = i * NBUF
            for b in range(NBUF):
                gather_wait(base + b, b)
                scatter_start(base + b, b)

            @pl.when(i + 1 < ngrp)
            def _():
                for b in range(NBUF):
                    scatter_wait(base + b, b)
                    gather_start(base + NBUF + b, b)

            return carry

        lax.fori_loop(0, ngrp, body, 0)
        for b in range(NBUF):
            scatter_wait(NCH - NBUF + b, b)
        plsc.subcore_barrier()
        pltpu.sync_copy(acc.at[pl.ds(r0, RPT)], out_hbm.at[c, pl.ds(r0, RPT)])

    return _sc_deg, _sc_agg


def _tc_h1_body(x_ref, w_ref, h_ref):
    h = jnp.dot(x_ref[...], w_ref[...], preferred_element_type=jnp.float32)
    h_ref[pl.ds(0, NN), :] = h
    h_ref[pl.ds(NN, NP - NN), :] = jnp.zeros((NP - NN, DH), jnp.float32)


_tc_h1 = pl.pallas_call(
    _tc_h1_body,
    out_shape=jax.ShapeDtypeStruct((NP, DH), jnp.float32),
)


def _tc_scale_body(deg0_ref, deg1_ref, h_ref, dis_ref, hs_ref):
    deg = deg0_ref[...] + deg1_ref[...] + 1.0
    dis = lax.rsqrt(deg)
    dis_ref[...] = dis
    hs_ref[...] = h_ref[...] * dis


_tc_scale = pl.pallas_call(
    _tc_scale_body,
    out_shape=(
        jax.ShapeDtypeStruct((NP, 1), jnp.float32),
        jax.ShapeDtypeStruct((NP, DH), jnp.float32),
    ),
)


def _tc_mid_body(ga_ref, gb_ref, hs_ref, dis_ref, b_ref, w_ref, out_ref):
    dis = dis_ref[...]
    u = jnp.maximum(dis * (ga_ref[...] + gb_ref[...] + hs_ref[...]) + b_ref[...], 0.0)
    hs2 = jnp.dot(u, w_ref[...], preferred_element_type=jnp.float32) * dis
    out_ref[...] = hs2
    out_ref[pl.ds(NN, NP - NN), :] = jnp.zeros((NP - NN, DH), jnp.float32)


_tc_mid = pl.pallas_call(
    _tc_mid_body,
    out_shape=jax.ShapeDtypeStruct((NP, DH), jnp.float32),
)


def _tc_out_body(ga_ref, gb_ref, hs_ref, dis_ref, b_ref, wl_ref, bl_ref, out_ref):
    dis = dis_ref[...]
    v = jnp.maximum(dis * (ga_ref[...] + gb_ref[...] + hs_ref[...]) + b_ref[...], 0.0)
    out = jnp.dot(v, wl_ref[...], preferred_element_type=jnp.float32) + bl_ref[...]
    out_ref[...] = lax.slice(out, (0, 0), (NN, 1))


_tc_out = pl.pallas_call(
    _tc_out_body,
    out_shape=jax.ShapeDtypeStruct((NN, 1), jnp.float32),
)


def kernel(x, edge_index, W1, b1, W2, b2, Wl, bl):
    src = edge_index[0]
    dst = edge_index[1]
    e = src.shape[0]
    pad = jnp.full((E2 - e,), NN, dtype=jnp.int32)
    srcp = jnp.concatenate([src, pad]).reshape(NCORE, NSUB, NCH, CHUNK)
    dstp = jnp.concatenate([dst, pad]).reshape(NCORE, NSUB, NCH, CHUNK)
    zeros1 = jnp.zeros((NP,), jnp.float32)
    zeros2 = jnp.zeros((NP, DH), jnp.float32)

    _sc_deg, _sc_agg = _sc_kernels()
    degp = _sc_deg(dstp, zeros1)
    h1 = _tc_h1(x, W1)
    dis, hs1 = _tc_scale(degp[0].reshape(NP, 1), degp[1].reshape(NP, 1), h1)
    g1 = _sc_agg(srcp, dstp, hs1, zeros2)
    hs2 = _tc_mid(g1[0], g1[1], hs1, dis, b1.reshape(1, DH), W2)
    g2 = _sc_agg(srcp, dstp, hs2, zeros2)
    out = _tc_out(g2[0], g2[1], hs2, dis, b2.reshape(1, DH), Wl, bl.reshape(1, 1))
    return out
